# Initial kernel scaffold; baseline (speedup 1.0000x reference)
import functools

import jax
import jax.numpy as jnp
from jax import lax
from jax.experimental import pallas as pl
from jax.experimental.pallas import tpu as pltpu
from jax.experimental.pallas import tpu_sc as plsc

NC = 2
NS = 16
NW = NC * NS
C = 128


@jax.jit
def _gather_flat(idx3, table):
    nw, n_chunks, c = idx3.shape
    V, D = table.shape
    N = nw * n_chunks * c
    b_per_w = n_chunks * c

    mesh = plsc.VectorSubcoreMesh(core_axis_name="c", subcore_axis_name="s")

    @functools.partial(
        pl.kernel,
        out_type=jax.ShapeDtypeStruct((N, D), jnp.float32),
        mesh=mesh,
        scratch_types=[
            pltpu.VMEM((n_chunks, c), jnp.int32),
            pltpu.VMEM((c, D), jnp.float32),
            pltpu.SemaphoreType.DMA,
        ],
    )
    def k(idx_hbm, table_hbm, out_hbm, idx_v, rows_v, gsem):
        wid = lax.axis_index("s") * NC + lax.axis_index("c")
        base = wid * b_per_w
        pltpu.sync_copy(idx_hbm.at[wid], idx_v)

        def body(j, _):
            pltpu.async_copy(table_hbm.at[idx_v.at[j]], rows_v, gsem).wait()
            pltpu.sync_copy(rows_v, out_hbm.at[pl.ds(base + j * c, c)])
            return ()

        lax.fori_loop(0, n_chunks, body, ())

    return k(idx3, table)


def kernel(inp, table):
    B, L = inp.shape
    D = table.shape[1]
    N = B * L
    idx3 = inp.reshape(NW, N // (NW * C), C).astype(jnp.int32)
    out = _gather_flat(idx3, table)
    return out.reshape(B, L, D)


# speedup vs baseline: 1.6864x; 1.6864x over previous
import functools

import jax
import jax.numpy as jnp
from jax import lax
from jax.experimental import pallas as pl
from jax.experimental.pallas import tpu as pltpu
from jax.experimental.pallas import tpu_sc as plsc

NC = 2
NS = 16
NW = NC * NS
C = 128


@jax.jit
def _gather_flat(idx3, table):
    nw, n_chunks, c = idx3.shape
    V, D = table.shape
    N = nw * n_chunks * c
    b_per_w = n_chunks * c

    mesh = plsc.VectorSubcoreMesh(core_axis_name="c", subcore_axis_name="s")

    @functools.partial(
        pl.kernel,
        out_type=jax.ShapeDtypeStruct((N, D), jnp.float32),
        mesh=mesh,
        scratch_types=[
            pltpu.VMEM((n_chunks, c), jnp.int32),
            pltpu.VMEM((c, D), jnp.float32),
            pltpu.SemaphoreType.DMA,
        ],
        compiler_params=pltpu.CompilerParams(use_tc_tiling_on_sc=False),
    )
    def k(idx_hbm, table_hbm, out_hbm, idx_v, rows_v, gsem):
        wid = lax.axis_index("s") * NC + lax.axis_index("c")
        base = wid * b_per_w
        pltpu.sync_copy(idx_hbm.at[wid], idx_v)

        def body(j, _):
            pltpu.async_copy(table_hbm.at[idx_v.at[j]], rows_v, gsem).wait()
            pltpu.sync_copy(rows_v, out_hbm.at[pl.ds(base + j * c, c)])
            return ()

        lax.fori_loop(0, n_chunks, body, ())

    return k(idx3, table)


def kernel(inp, table):
    B, L = inp.shape
    D = table.shape[1]
    N = B * L
    idx3 = inp.reshape(NW, N // (NW * C), C).astype(jnp.int32)
    out = _gather_flat(idx3, table)
    return out.reshape(B, L, D)


# speedup vs baseline: 1.8738x; 1.1112x over previous
import functools

import jax
import jax.numpy as jnp
from jax import lax
from jax.experimental import pallas as pl
from jax.experimental.pallas import tpu as pltpu
from jax.experimental.pallas import tpu_sc as plsc

NC = 2
NS = 16
NW = NC * NS
C = 128
NBUF = 8
LOOKA = 6


@jax.jit
def _gather_flat(idx3, table):
    nw, n_chunks, c = idx3.shape
    V, D = table.shape
    N = nw * n_chunks * c
    b_per_w = n_chunks * c

    mesh = plsc.VectorSubcoreMesh(core_axis_name="c", subcore_axis_name="s")

    @functools.partial(
        pl.kernel,
        out_type=jax.ShapeDtypeStruct((N, D), jnp.float32),
        mesh=mesh,
        scratch_types=[
            pltpu.VMEM((n_chunks, c), jnp.int32),
            pltpu.VMEM((NBUF, c, D), jnp.float32),
            pltpu.SemaphoreType.DMA((NBUF,)),
            pltpu.SemaphoreType.DMA((NBUF,)),
        ],
        compiler_params=pltpu.CompilerParams(use_tc_tiling_on_sc=False),
    )
    def k(idx_hbm, table_hbm, out_hbm, idx_v, rows_v, gsem, osem):
        wid = lax.axis_index("s") * NC + lax.axis_index("c")
        base = wid * b_per_w
        pltpu.sync_copy(idx_hbm.at[wid], idx_v)

        def start_gather(j, b):
            pltpu.async_copy(table_hbm.at[idx_v.at[j]], rows_v.at[b],
                             gsem.at[b])

        def wait_gather(j, b):
            pltpu.make_async_copy(table_hbm.at[idx_v.at[j]], rows_v.at[b],
                                  gsem.at[b]).wait()

        def start_write(j, b):
            pltpu.async_copy(rows_v.at[b], out_hbm.at[pl.ds(base + j * c, c)],
                             osem.at[b])

        def wait_write(j, b):
            pltpu.make_async_copy(rows_v.at[b],
                                  out_hbm.at[pl.ds(base + j * c, c)],
                                  osem.at[b]).wait()

        for b in range(LOOKA):
            start_gather(b, b)

        def body(jj):
            for b in range(NBUF):
                j = jj + b
                b2 = (b + LOOKA) % NBUF
                j2 = j + LOOKA
                wait_gather(j, b)
                start_write(j, b)

                @pl.when((j2 < n_chunks) & (j2 >= NBUF))
                def _():
                    wait_write(j2 - NBUF, b2)

                @pl.when(j2 < n_chunks)
                def _():
                    start_gather(j2, b2)

        pl.loop(0, n_chunks, step=NBUF)(body)

        for b in range(NBUF):
            wait_write(n_chunks - NBUF + b, b)

    return k(idx3, table)


def kernel(inp, table):
    B, L = inp.shape
    D = table.shape[1]
    N = B * L
    idx3 = inp.reshape(NW, N // (NW * C), C).astype(jnp.int32)
    out = _gather_flat(idx3, table)
    return out.reshape(B, L, D)
